# Initial kernel scaffold; baseline (speedup 1.0000x reference)
import jax
import jax.numpy as jnp
from jax.experimental import pallas as pl


def kernel(center, nbrs, negs, embedI_w, embedO_w):
    raise NotImplementedError("write your pallas kernel here")


# speedup vs baseline: 5.3312x; 5.3312x over previous
import functools

import jax
import jax.numpy as jnp
from jax import lax
from jax.experimental import pallas as pl
from jax.experimental.pallas import tpu as pltpu
from jax.experimental.pallas import tpu_sc as plsc

_DIMV = 1000000
_E = 64
_B = 16384
_K = 20
_NC = 2
_NS = 16
_NW = _NC * _NS
_NB = _B // _NW
_C = 32
_CK = _C * _K
_GCH = 128
_NG = _CK // _GCH
_L = 16


def _sc_body(center, nbrs, negsf, embedI, embedO, uv_out, sn_out,
             cidx, nidx, gidx, vbuf, ubuf, negbuf, uvv, snv, sem):
    wid = lax.axis_index("s") * _NC + lax.axis_index("c")
    base = wid * _NB

    pltpu.sync_copy(center.at[pl.ds(base, _NB)], cidx)
    pltpu.sync_copy(nbrs.at[pl.ds(base, _NB)], nidx)
    pltpu.sync_copy(negsf.at[pl.ds(base * _K, _NB * _K)], gidx)

    def _copies(it):
        c0 = it * _C
        ops = [
            pltpu.make_async_copy(embedI.at[cidx.at[pl.ds(c0, _C)]], vbuf, sem),
            pltpu.make_async_copy(embedO.at[nidx.at[pl.ds(c0, _C)]], ubuf, sem),
        ]
        for j in range(_NG):
            ops.append(pltpu.make_async_copy(
                embedO.at[gidx.at[pl.ds(c0 * _K + j * _GCH, _GCH)]],
                negbuf.at[pl.ds(j * _GCH, _GCH)], sem))
        return ops

    def _chunk(it, carry):
        ops = _copies(it)
        for o in ops:
            o.start()
        for o in ops:
            o.wait()

        def _elem(c, carry2):
            vv = [vbuf[c, pl.ds(j * _L, _L)] for j in range(4)]
            uu = [ubuf[c, pl.ds(j * _L, _L)] for j in range(4)]
            uvacc = (vv[0] * uu[0] + vv[1] * uu[1]) + (vv[2] * uu[2] + vv[3] * uu[3])
            accs = [negbuf[c * _K, pl.ds(j * _L, _L)] for j in range(4)]
            for k in range(1, _K):
                r = c * _K + k
                for j in range(4):
                    accs[j] = accs[j] + negbuf[r, pl.ds(j * _L, _L)]
            snacc = (accs[0] * vv[0] + accs[1] * vv[1]) + (accs[2] * vv[2] + accs[3] * vv[3])
            uvv[it * _C + c, :] = uvacc
            snv[it * _C + c, :] = snacc
            return carry2

        lax.fori_loop(0, _C, _elem, 0)
        return carry

    lax.fori_loop(0, _NB // _C, _chunk, 0)

    pltpu.sync_copy(uvv, uv_out.at[pl.ds(base, _NB)])
    pltpu.sync_copy(snv, sn_out.at[pl.ds(base, _NB)])


_sc_dots = functools.partial(
    pl.kernel,
    out_type=[jax.ShapeDtypeStruct((_B, _L), jnp.float32),
              jax.ShapeDtypeStruct((_B, _L), jnp.float32)],
    mesh=plsc.VectorSubcoreMesh(core_axis_name="c", subcore_axis_name="s"),
    compiler_params=pltpu.CompilerParams(use_tc_tiling_on_sc=False),
    scratch_types=[
        pltpu.VMEM((_NB,), jnp.int32),
        pltpu.VMEM((_NB,), jnp.int32),
        pltpu.VMEM((_NB * _K,), jnp.int32),
        pltpu.VMEM((_C, _E), jnp.float32),
        pltpu.VMEM((_C, _E), jnp.float32),
        pltpu.VMEM((_CK, _E), jnp.float32),
        pltpu.VMEM((_NB, _L), jnp.float32),
        pltpu.VMEM((_NB, _L), jnp.float32),
        pltpu.SemaphoreType.DMA,
    ],
)(_sc_body)


def _log_sigmoid(x):
    return jnp.minimum(x, 0.0) - jnp.log1p(jnp.exp(-jnp.abs(x)))


def _loss_body(uv_ref, sn_ref, out_ref):
    qi = lax.broadcasted_iota(jnp.int32, (2048, 128), 0)
    ci = lax.broadcasted_iota(jnp.int32, (2048, 128), 1)
    comb = jnp.where(qi // _L == ci, 1.0, 0.0).astype(jnp.float32)
    uv = jnp.dot(uv_ref[...], comb, preferred_element_type=jnp.float32)
    sn = jnp.dot(sn_ref[...], comb, preferred_element_type=jnp.float32)
    pos = _log_sigmoid(uv)
    neg = _log_sigmoid(-sn)
    out_ref[...] = -(jnp.sum(pos, keepdims=True) + jnp.sum(neg, keepdims=True)) / _B


def kernel(center, nbrs, negs, embedI_w, embedO_w):
    center = center.astype(jnp.int32)
    nbrs = nbrs.astype(jnp.int32)
    negsf = negs.astype(jnp.int32).reshape(-1)
    uv, sn = _sc_dots(center, nbrs, negsf, embedI_w, embedO_w)
    out = pl.pallas_call(
        _loss_body,
        out_shape=jax.ShapeDtypeStruct((1, 1), jnp.float32),
    )(uv.reshape(128, 2048), sn.reshape(128, 2048))
    return out[0, 0]
